# baseline (speedup 1.0000x reference)
import functools

import jax
import jax.numpy as jnp
from jax import lax
from jax.experimental import pallas as pl
from jax.experimental.pallas import tpu as pltpu
from jax.experimental.pallas import tpu_sc as plsc

NC = 2
NS = 16
NW = NC * NS
CH = 128
NBUF = 4


def _sc_embedding_bag(B, G, D, center_ids, ngram_ids, W_in, W_sub):
    b_per_w = B // NW
    n_ctr = b_per_w // CH
    n_sub = (b_per_w * G) // CH
    n_sub_p = n_sub + NBUF
    dummy = NS * b_per_w

    mesh = plsc.VectorSubcoreMesh(core_axis_name="c", subcore_axis_name="s")

    @functools.partial(
        pl.kernel,
        mesh=mesh,
        out_type=jax.ShapeDtypeStruct((B, D), jnp.float32),
        compiler_params=pltpu.CompilerParams(
            use_tc_tiling_on_sc=False, needs_layout_passes=False),
        scratch_types=dict(
            ctr_v=pltpu.VMEM((b_per_w,), jnp.int32),
            ngraw_v=pltpu.VMEM((b_per_w, G), jnp.int32),
            ng_v=pltpu.VMEM((n_sub_p, CH), jnp.int32),
            slot_v=pltpu.VMEM((n_sub_p, CH), jnp.int32),
            bufs=pltpu.VMEM((NBUF, CH, D), jnp.float32),
            acc=pltpu.VMEM_SHARED((NS * b_per_w + 8, D), jnp.float32),
            gsems=pltpu.SemaphoreType.DMA((NBUF,)),
            sem=pltpu.SemaphoreType.DMA,
        ),
    )
    def k(ctr_hbm, ng_hbm, w_in, w_sub, out,
          ctr_v, ngraw_v, ng_v, slot_v, bufs, acc, gsems, sem):
        cid = lax.axis_index("c")
        sid = lax.axis_index("s")
        wid = sid * NC + cid
        base = wid * b_per_w
        abase = sid * b_per_w

        pltpu.sync_copy(ctr_hbm.at[pl.ds(base, b_per_w)], ctr_v)
        pltpu.sync_copy(ng_hbm.at[pl.ds(base, b_per_w), :], ngraw_v)

        def repack(j, carry):
            for m in range(CH // 16):
                p = lax.iota(jnp.int32, 16) + jnp.full(
                    16, j * CH + m * 16, jnp.int32)
                q = lax.div(p, jnp.full(16, G, jnp.int32))

                @pl.when(j < n_sub)
                def _():
                    r = p - q * jnp.full(16, G, jnp.int32)
                    ng_v[j, pl.ds(m * 16, 16)] = plsc.load_gather(
                        ngraw_v, [q, r])
                    slot_v[j, pl.ds(m * 16, 16)] = q + jnp.full(
                        16, abase, jnp.int32)

                @pl.when(j >= n_sub)
                def _():
                    ng_v[j, pl.ds(m * 16, 16)] = jnp.zeros(16, jnp.int32)
                    slot_v[j, pl.ds(m * 16, 16)] = jnp.full(
                        16, dummy, jnp.int32)
            return carry

        lax.fori_loop(0, n_sub_p, repack, 0)

        for c in range(n_ctr):
            pltpu.async_copy(
                w_in.at[ctr_v.at[pl.ds(c * CH, CH)]], bufs.at[0], sem).wait()
            pltpu.sync_copy(bufs.at[0], acc.at[pl.ds(abase + c * CH, CH)])

        for b in range(NBUF):
            pltpu.async_copy(w_sub.at[ng_v.at[b]], bufs.at[b], gsems.at[b])

        def step(i, carry):
            c = i * NBUF
            for b in range(NBUF):
                pltpu.make_async_copy(
                    w_sub.at[ng_v.at[c + b]], bufs.at[b], gsems.at[b]).wait()
                pltpu.sync_copy(bufs.at[b], acc.at[slot_v.at[c + b]], add=True)
                pltpu.async_copy(
                    w_sub.at[ng_v.at[c + b + NBUF]], bufs.at[b], gsems.at[b])
            return carry

        lax.fori_loop(0, n_sub // NBUF, step, 0)
        for b in range(NBUF):
            pltpu.make_async_copy(
                w_sub.at[ng_v.at[n_sub + b]], bufs.at[b], gsems.at[b]).wait()

        pltpu.sync_copy(acc.at[pl.ds(abase, b_per_w)],
                        out.at[pl.ds(base, b_per_w)])

    return k(center_ids, ngram_ids, W_in, W_sub)


def kernel(center_ids, ngram_ids, W_in, W_sub):
    B, G = ngram_ids.shape
    D = W_in.shape[1]
    return _sc_embedding_bag(
        B, G, D,
        center_ids.astype(jnp.int32), ngram_ids.astype(jnp.int32),
        W_in, W_sub)


# speedup vs baseline: 1.1153x; 1.1153x over previous
import functools

import jax
import jax.numpy as jnp
from jax import lax
from jax.experimental import pallas as pl
from jax.experimental.pallas import tpu as pltpu
from jax.experimental.pallas import tpu_sc as plsc

NC = 2
NS = 16
NW = NC * NS
CH = 128
NBUF = 2
NPASS = 4
L = 16


def _sc_embedding_bag(B, G, D, center_ids, ngT, win_p, wsub_p):
    b_per_w = B // NW
    pw = b_per_w // NPASS
    n_chunks = 1 + G
    W2 = 2 * D

    mesh = plsc.VectorSubcoreMesh(core_axis_name="c", subcore_axis_name="s")

    @functools.partial(
        pl.kernel,
        mesh=mesh,
        out_type=jax.ShapeDtypeStruct((B * D,), jnp.float32),
        compiler_params=pltpu.CompilerParams(
            use_tc_tiling_on_sc=True, needs_layout_passes=False),
        scratch_types=dict(
            ctr_v=pltpu.VMEM((b_per_w,), jnp.int32),
            ngT_v=pltpu.VMEM((G, b_per_w), jnp.int32),
            pidx=pltpu.VMEM((NBUF, CH), jnp.int32),
            slot=pltpu.VMEM((NBUF, CH), jnp.int32),
            bufs=pltpu.VMEM((NBUF, CH, W2), jnp.float32),
            vbuf=pltpu.VMEM((2 * pw, W2), jnp.float32),
            obuf=pltpu.VMEM((pw * D,), jnp.float32),
            acc2=pltpu.VMEM_SHARED((NS * 2 * pw, W2), jnp.float32),
            gsems=pltpu.SemaphoreType.DMA((NBUF,)),
        ),
    )
    def k(ctr_hbm, ngT_hbm, win, wsub, out,
          ctr_v, ngT_v, pidx, slot, bufs, vbuf, obuf, acc2, gsems):
        cid = lax.axis_index("c")
        sid = lax.axis_index("s")
        wid = sid * NC + cid
        base = wid * b_per_w
        abase = sid * 2 * pw
        io16 = lax.iota(jnp.int32, L)
        one = jnp.full(L, 1, jnp.int32)

        pltpu.sync_copy(ctr_hbm.at[pl.ds(base, b_per_w)], ctr_v)
        pltpu.sync_copy(ngT_hbm.at[:, pl.ds(base, b_per_w)], ngT_v)

        def emit(b, m, ids, wbase):
            par = lax.bitwise_and(ids, one)
            w16 = io16 + jnp.full(L, m * L, jnp.int32) + wbase
            off = pl.multiple_of(m * L, L)
            pidx[b, pl.ds(off, L)] = lax.shift_right_logical(ids, one)
            slot[b, pl.ds(off, L)] = (
                w16 + w16 + one - par + jnp.full(L, abase, jnp.int32))

        def fire(b, tab):
            return pltpu.async_copy(
                tab.at[pidx.at[b]], bufs.at[b], gsems.at[b])

        zero16 = jnp.zeros(L, jnp.int32)

        for p in range(NPASS):
            woff = p * pw

            def zv(kk, carry):
                r = lax.shift_right_logical(kk, 3)
                m = lax.bitwise_and(kk, 7)
                off = pl.multiple_of(m * L, L)
                vbuf[r, pl.ds(off, L)] = jnp.zeros(L, jnp.float32)
                return carry

            lax.fori_loop(0, 2 * pw * (W2 // L), zv, 0)
            pltpu.sync_copy(vbuf, acc2.at[pl.ds(abase, 2 * pw)])

            for m in range(CH // L):
                emit(0, m, ctr_v[pl.ds(woff + m * L, L)], zero16)
            fire(0, win)
            for m in range(CH // L):
                emit(1, m, ngT_v[0, pl.ds(woff + m * L, L)], zero16)
            fire(1, wsub)

            def step(t, carry):
                bb = lax.rem(t, NBUF)
                for b in range(NBUF):
                    @pl.when(bb == b)
                    def _():
                        pltpu.make_async_copy(
                            wsub.at[pidx.at[b]], bufs.at[b],
                            gsems.at[b]).wait()
                        pltpu.sync_copy(
                            bufs.at[b], acc2.at[slot.at[b]], add=True)

                        @pl.when(t + NBUF < n_chunks)
                        def _():
                            g = t + NBUF - 1
                            for m in range(CH // L):
                                emit(b, m,
                                     ngT_v[g, pl.ds(pl.multiple_of(
                                         woff + m * L, L), L)],
                                     zero16)
                            fire(b, wsub)
                return carry

            lax.fori_loop(0, n_chunks, step, 0)

            pltpu.sync_copy(acc2.at[pl.ds(abase, 2 * pw)], vbuf)

            def fin(w, carry):
                w2 = w + w
                for m in range(D // L):
                    off = pl.multiple_of(m * L, L)
                    obuf[pl.ds(w * D + off, L)] = (
                        vbuf[w2 + 1, pl.ds(off, L)]
                        + vbuf[w2, pl.ds(pl.multiple_of(D + m * L, L), L)])
                return carry

            lax.fori_loop(0, pw, fin, 0)
            pltpu.sync_copy(
                obuf, out.at[pl.ds((base + woff) * D, pw * D)])

    return k(center_ids, ngT, win_p, wsub_p)


def kernel(center_ids, ngram_ids, W_in, W_sub):
    B, G = ngram_ids.shape
    D = W_in.shape[1]
    ngT = jnp.swapaxes(ngram_ids.astype(jnp.int32), 0, 1)
    win_p = W_in.reshape(-1, 2 * D)
    wsub_p = W_sub.reshape(-1, 2 * D)
    out = _sc_embedding_bag(
        B, G, D, center_ids.astype(jnp.int32), ngT, win_p, wsub_p)
    return out.reshape(B, D)


# speedup vs baseline: 1.1840x; 1.0616x over previous
import functools

import jax
import jax.numpy as jnp
from jax import lax
from jax.experimental import pallas as pl
from jax.experimental.pallas import tpu as pltpu
from jax.experimental.pallas import tpu_sc as plsc

NC = 2
NS = 16
NW = NC * NS
CH = 128
NBUF = 2
NPASS = 4
L = 16

_mesh = lambda: plsc.VectorSubcoreMesh(core_axis_name="c",
                                       subcore_axis_name="s")
_params = pltpu.CompilerParams(
    use_tc_tiling_on_sc=True, needs_layout_passes=False)


def _center_lookup(B, D, center_ids, win_p):
    b_per_w = B // NW
    n_ctr = b_per_w // CH
    W2 = 2 * D

    @functools.partial(
        pl.kernel,
        mesh=_mesh(),
        out_type=jax.ShapeDtypeStruct((B * D,), jnp.float32),
        compiler_params=_params,
        scratch_types=dict(
            ctr_v=pltpu.VMEM((b_per_w,), jnp.int32),
            bufs=pltpu.VMEM((NBUF, CH, W2), jnp.float32),
            obuf=pltpu.VMEM((CH * D,), jnp.float32),
            gsems=pltpu.SemaphoreType.DMA((NBUF,)),
        ),
    )
    def k1(ctr_hbm, win, out, ctr_v, bufs, obuf, gsems):
        cid = lax.axis_index("c")
        sid = lax.axis_index("s")
        wid = sid * NC + cid
        base = wid * b_per_w

        pltpu.sync_copy(ctr_hbm.at[pl.ds(base, b_per_w)], ctr_v)
        for b in range(NBUF):
            pltpu.async_copy(
                win.at[ctr_v.at[pl.ds(b * CH, CH)]], bufs.at[b],
                gsems.at[b])
        for c in range(n_ctr):
            b = c % NBUF
            pltpu.make_async_copy(
                win.at[ctr_v.at[pl.ds(c * CH, CH)]], bufs.at[b],
                gsems.at[b]).wait()

            def cp(w, carry):
                for m in range(D // L):
                    off = pl.multiple_of(m * L, L)
                    obuf[pl.ds(w * D + off, L)] = bufs[b, w, pl.ds(off, L)]
                return carry

            lax.fori_loop(0, CH, cp, 0)
            pltpu.sync_copy(
                obuf, out.at[pl.ds((base + c * CH) * D, CH * D)])
            if c + NBUF < n_ctr:
                pltpu.async_copy(
                    win.at[ctr_v.at[pl.ds((c + NBUF) * CH, CH)]],
                    bufs.at[b], gsems.at[b])

    return k1(center_ids, win_p)


def _ngram_sum(B, G, D, ngT, wsub_p, partial):
    b_per_w = B // NW
    pw = b_per_w // NPASS
    W2 = 2 * D

    @functools.partial(
        pl.kernel,
        mesh=_mesh(),
        out_type=jax.ShapeDtypeStruct((B * D,), jnp.float32),
        compiler_params=_params,
        scratch_types=dict(
            ngT_v=pltpu.VMEM((G, b_per_w), jnp.int32),
            pidx=pltpu.VMEM((NBUF, CH), jnp.int32),
            slot=pltpu.VMEM((NBUF, CH), jnp.int32),
            bufs=pltpu.VMEM((NBUF, CH, W2), jnp.float32),
            vbuf=pltpu.VMEM((pw, W2), jnp.float32),
            pbuf=pltpu.VMEM((pw * D,), jnp.float32),
            obuf=pltpu.VMEM((pw * D,), jnp.float32),
            acc2=pltpu.VMEM_SHARED((NS * pw, W2), jnp.float32),
            gsems=pltpu.SemaphoreType.DMA((NBUF,)),
            psem=pltpu.SemaphoreType.DMA,
        ),
    )
    def k2(ngT_hbm, wsub, part, out,
           ngT_v, pidx, slot, bufs, vbuf, pbuf, obuf, acc2, gsems, psem):
        cid = lax.axis_index("c")
        sid = lax.axis_index("s")
        wid = sid * NC + cid
        base = wid * b_per_w
        abase = sid * pw
        io16 = lax.iota(jnp.int32, L)

        pltpu.sync_copy(ngT_hbm.at[:, pl.ds(base, b_per_w)], ngT_v)

        def emit(b, m, ids):
            off = pl.multiple_of(m * L, L)
            pidx[b, pl.ds(off, L)] = ids
            slot[b, pl.ds(off, L)] = io16 + jnp.full(
                L, abase + m * L, jnp.int32)

        def fire(b):
            return pltpu.async_copy(
                wsub.at[pidx.at[b]], bufs.at[b], gsems.at[b])

        for p in range(NPASS):
            woff = p * pw

            pcp = pltpu.async_copy(
                part.at[pl.ds((base + woff) * D, pw * D)], pbuf, psem)

            def zv(kk, carry):
                r = lax.shift_right_logical(kk, 3)
                m = lax.bitwise_and(kk, 7)
                off = pl.multiple_of(m * L, L)
                vbuf[r, pl.ds(off, L)] = jnp.zeros(L, jnp.float32)
                return carry

            lax.fori_loop(0, pw * (W2 // L), zv, 0)
            pltpu.sync_copy(vbuf, acc2.at[pl.ds(abase, pw)])

            for b in range(NBUF):
                for m in range(CH // L):
                    emit(b, m, ngT_v[b, pl.ds(woff + m * L, L)])
                fire(b)

            def step(t, carry):
                bb = lax.rem(t, NBUF)
                for b in range(NBUF):
                    @pl.when(bb == b)
                    def _():
                        pltpu.make_async_copy(
                            wsub.at[pidx.at[b]], bufs.at[b],
                            gsems.at[b]).wait()
                        pltpu.sync_copy(
                            bufs.at[b], acc2.at[slot.at[b]], add=True)

                        @pl.when(t + NBUF < G)
                        def _():
                            g = t + NBUF
                            for m in range(CH // L):
                                emit(b, m,
                                     ngT_v[g, pl.ds(pl.multiple_of(
                                         woff + m * L, L), L)])
                            fire(b)
                return carry

            lax.fori_loop(0, G, step, 0)

            pltpu.sync_copy(acc2.at[pl.ds(abase, pw)], vbuf)
            pcp.wait()

            def fin(w, carry):
                for m in range(D // L):
                    off = pl.multiple_of(m * L, L)
                    po = pl.multiple_of(w * D + off, L)
                    obuf[pl.ds(po, L)] = (
                        vbuf[w, pl.ds(off, L)] + pbuf[pl.ds(po, L)])
                return carry

            lax.fori_loop(0, pw, fin, 0)
            pltpu.sync_copy(
                obuf, out.at[pl.ds((base + woff) * D, pw * D)])

    return k2(ngT, wsub_p, partial)


def kernel(center_ids, ngram_ids, W_in, W_sub):
    B, G = ngram_ids.shape
    D = W_in.shape[1]
    ngT = jnp.swapaxes(ngram_ids.astype(jnp.int32), 0, 1)
    win_p = jnp.pad(W_in, ((0, 0), (0, D)))
    wsub_p = jnp.pad(W_sub, ((0, 0), (0, D)))
    partial = _center_lookup(B, D, center_ids.astype(jnp.int32), win_p)
    out = _ngram_sum(B, G, D, ngT, wsub_p, partial)
    return out.reshape(B, D)


# speedup vs baseline: 1.1846x; 1.0005x over previous
import functools

import jax
import jax.numpy as jnp
from jax import lax
from jax.experimental import pallas as pl
from jax.experimental.pallas import tpu as pltpu
from jax.experimental.pallas import tpu_sc as plsc

NC = 2
NS = 16
NW = NC * NS
CH = 128
NBUF = 2
NPASS = 4
L = 16


def _sc_embedding_bag(B, G, D, center_ids, ngT, win_p, wsub_p):
    b_per_w = B // NW
    pw = b_per_w // NPASS
    n_chunks = 1 + G
    W2 = 2 * D

    mesh = plsc.VectorSubcoreMesh(core_axis_name="c", subcore_axis_name="s")

    @functools.partial(
        pl.kernel,
        mesh=mesh,
        out_type=jax.ShapeDtypeStruct((B * D,), jnp.float32),
        compiler_params=pltpu.CompilerParams(
            use_tc_tiling_on_sc=True, needs_layout_passes=False),
        scratch_types=dict(
            ctr_v=pltpu.VMEM((b_per_w,), jnp.int32),
            ngT_v=pltpu.VMEM((G, b_per_w), jnp.int32),
            pidx=pltpu.VMEM((NBUF, CH), jnp.int32),
            slot=pltpu.VMEM((NBUF, CH), jnp.int32),
            bufs=pltpu.VMEM((NBUF, CH, W2), jnp.float32),
            vbuf=pltpu.VMEM((pw, W2), jnp.float32),
            obuf=pltpu.VMEM((pw * D,), jnp.float32),
            acc2=pltpu.VMEM_SHARED((NS * pw, W2), jnp.float32),
            gsems=pltpu.SemaphoreType.DMA((NBUF,)),
        ),
    )
    def k(ctr_hbm, ngT_hbm, win, wsub, out,
          ctr_v, ngT_v, pidx, slot, bufs, vbuf, obuf, acc2, gsems):
        cid = lax.axis_index("c")
        sid = lax.axis_index("s")
        wid = sid * NC + cid
        base = wid * b_per_w
        abase = sid * pw
        io16 = lax.iota(jnp.int32, L)
        one = jnp.full(L, 1, jnp.int32)

        pltpu.sync_copy(ctr_hbm.at[pl.ds(base, b_per_w)], ctr_v)
        pltpu.sync_copy(ngT_hbm.at[:, pl.ds(base, b_per_w)], ngT_v)

        def emit(b, m, ids, wbase):
            w16 = io16 + jnp.full(L, m * L, jnp.int32) + wbase
            off = pl.multiple_of(m * L, L)
            pidx[b, pl.ds(off, L)] = ids
            slot[b, pl.ds(off, L)] = w16 + jnp.full(L, abase, jnp.int32)

        def fire(b, tab):
            return pltpu.async_copy(
                tab.at[pidx.at[b]], bufs.at[b], gsems.at[b])

        zero16 = jnp.zeros(L, jnp.int32)

        for p in range(NPASS):
            woff = p * pw

            def zv(kk, carry):
                r = lax.shift_right_logical(kk, 3)
                m = lax.bitwise_and(kk, 7)
                off = pl.multiple_of(m * L, L)
                vbuf[r, pl.ds(off, L)] = jnp.zeros(L, jnp.float32)
                return carry

            lax.fori_loop(0, pw * (W2 // L), zv, 0)
            pltpu.sync_copy(vbuf, acc2.at[pl.ds(abase, pw)])

            for m in range(CH // L):
                emit(0, m, ctr_v[pl.ds(woff + m * L, L)], zero16)
            fire(0, win)
            for m in range(CH // L):
                emit(1, m, ngT_v[0, pl.ds(woff + m * L, L)], zero16)
            fire(1, wsub)

            def step(t, carry):
                bb = lax.rem(t, NBUF)
                for b in range(NBUF):
                    @pl.when(bb == b)
                    def _():
                        pltpu.make_async_copy(
                            wsub.at[pidx.at[b]], bufs.at[b],
                            gsems.at[b]).wait()
                        pltpu.sync_copy(
                            bufs.at[b], acc2.at[slot.at[b]], add=True)

                        @pl.when(t + NBUF < n_chunks)
                        def _():
                            g = t + NBUF - 1
                            for m in range(CH // L):
                                emit(b, m,
                                     ngT_v[g, pl.ds(pl.multiple_of(
                                         woff + m * L, L), L)],
                                     zero16)
                            fire(b, wsub)
                return carry

            lax.fori_loop(0, n_chunks, step, 0)

            pltpu.sync_copy(acc2.at[pl.ds(abase, pw)], vbuf)

            def fin(w, carry):
                for m in range(D // L):
                    off = pl.multiple_of(m * L, L)
                    obuf[pl.ds(w * D + off, L)] = vbuf[w, pl.ds(off, L)]
                return carry

            lax.fori_loop(0, pw, fin, 0)
            pltpu.sync_copy(
                obuf, out.at[pl.ds((base + woff) * D, pw * D)])

    return k(center_ids, ngT, win_p, wsub_p)


def kernel(center_ids, ngram_ids, W_in, W_sub):
    B, G = ngram_ids.shape
    D = W_in.shape[1]
    ngT = jnp.swapaxes(ngram_ids.astype(jnp.int32), 0, 1)
    win_p = jnp.pad(W_in, ((0, 0), (0, D)))
    wsub_p = jnp.pad(W_sub, ((0, 0), (0, D)))
    out = _sc_embedding_bag(
        B, G, D, center_ids.astype(jnp.int32), ngT, win_p, wsub_p)
    return out.reshape(B, D)
